# Initial kernel scaffold; baseline (speedup 1.0000x reference)
import jax
import jax.numpy as jnp
from jax.experimental import pallas as pl


def kernel(x, edge_index, weights):
    raise NotImplementedError("write your pallas kernel here")


# speedup vs baseline: 5.7348x; 5.7348x over previous
import functools

import jax
import jax.numpy as jnp
from jax import lax
from jax.experimental import pallas as pl
from jax.experimental.pallas import tpu as pltpu
from jax.experimental.pallas import tpu_sc as plsc

NC = 2
NS = 16
NL = 16
NW = NC * NS
K = 64


def _mesh():
    return plsc.VectorSubcoreMesh(core_axis_name="c", subcore_axis_name="s",
                                  num_cores=NC, num_subcores=NS)


@functools.partial(jax.jit, static_argnames=("R", "C"))
def _deg_call(dstw, *, R, C):
    RS = R // NS

    @functools.partial(
        pl.kernel,
        out_type=jax.ShapeDtypeStruct((NC, R, NL), jnp.float32),
        mesh=_mesh(),
        scratch_types=[
            pltpu.VMEM_SHARED((R, NL), jnp.float32),
            pltpu.VMEM((K,), jnp.int32),
            pltpu.VMEM((K, NL), jnp.float32),
            pltpu.VMEM((K, NL), jnp.float32),
        ],
    )
    def deg_kernel(dstw_hbm, part_hbm, acc, dst_i, ones, zeros):
        cid = lax.axis_index("c")
        sid = lax.axis_index("s")
        wid = cid * NS + sid

        def fill(i, _):
            ones[i] = jnp.full((NL,), 1.0, jnp.float32)
            zeros[i] = jnp.full((NL,), 0.0, jnp.float32)
            return 0

        lax.fori_loop(0, K, fill, 0)

        row0 = sid * RS
        for q in range(RS // K):
            pltpu.sync_copy(zeros, acc.at[pl.ds(row0 + q * K, K)])
        rem = RS % K
        if rem:
            pltpu.sync_copy(zeros.at[pl.ds(0, rem)],
                            acc.at[pl.ds(row0 + (RS // K) * K, rem)])
        plsc.subcore_barrier()

        def body(j, _):
            pltpu.sync_copy(dstw_hbm.at[wid, j], dst_i)
            pltpu.sync_copy(ones, acc.at[dst_i], add=True)
            return 0

        lax.fori_loop(0, C, body, 0)
        plsc.subcore_barrier()
        pltpu.sync_copy(acc.at[pl.ds(row0, RS)],
                        part_hbm.at[cid, pl.ds(row0, RS)])

    return deg_kernel(dstw)


@functools.partial(jax.jit, static_argnames=("R", "C"))
def _prop_call(v, srcw, dstw, *, R, C):
    D = v.shape[1]
    RS = R // NS
    T = C // 4

    @functools.partial(
        pl.kernel,
        out_type=jax.ShapeDtypeStruct((NC, R, D), jnp.float32),
        mesh=_mesh(),
        scratch_types=(
            [pltpu.VMEM_SHARED((R, D), jnp.float32)]
            + [pltpu.VMEM((K,), jnp.int32)] * 8
            + [pltpu.VMEM((K, D), jnp.float32)] * 4
            + [pltpu.SemaphoreType.DMA] * 12
        ),
    )
    def prop_kernel(v_hbm, srcw_hbm, dstw_hbm, part_hbm, acc, *bufs):
        si = bufs[0:4]
        di = bufs[4:8]
        rr = bufs[8:12]
        sg = bufs[12:16]
        ss = bufs[16:20]
        sd = bufs[20:24]
        cid = lax.axis_index("c")
        sid = lax.axis_index("s")
        wid = cid * NS + sid

        def zfill(i, _):
            for c in range(D // NL):
                rr[0][i, pl.ds(c * NL, NL)] = jnp.full((NL,), 0.0, jnp.float32)
            return 0

        lax.fori_loop(0, K, zfill, 0)
        row0 = sid * RS
        for q in range(RS // K):
            pltpu.sync_copy(rr[0], acc.at[pl.ds(row0 + q * K, K)])
        rem = RS % K
        if rem:
            pltpu.sync_copy(rr[0].at[pl.ds(0, rem)],
                            acc.at[pl.ds(row0 + (RS // K) * K, rem)])
        plsc.subcore_barrier()

        def gfire(b, q):
            pltpu.async_copy(v_hbm.at[si[b]], rr[b], sg[b])

        def gwait(b):
            pltpu.make_async_copy(v_hbm.at[si[b]], rr[b], sg[b]).wait()

        def ifire(hbm, q, buf, sem):
            pltpu.async_copy(hbm.at[wid, q], buf, sem)

        def iwait(hbm, q, buf, sem):
            pltpu.make_async_copy(hbm.at[wid, q], buf, sem).wait()

        pltpu.sync_copy(srcw_hbm.at[wid, 0], si[0])
        ifire(dstw_hbm, 0, di[0], sd[0])
        ifire(srcw_hbm, 1, si[1], ss[1])
        ifire(dstw_hbm, 1, di[1], sd[1])
        gfire(0, 0)

        T_ = C

        def body_final(t4, _):
            for b in range(4):
                q = 4 * t4 + b
                bn = (b + 1) % 4
                bnn = (b + 2) % 4

                @pl.when(q + 1 < T_)
                def _(bn=bn, q=q):
                    iwait(srcw_hbm, q + 1, si[bn], ss[bn])
                    gfire(bn, q + 1)

                iwait(dstw_hbm, q, di[b], sd[b])
                gwait(b)
                pltpu.sync_copy(rr[b], acc.at[di[b]], add=True)

                @pl.when(q + 2 < T_)
                def _(bnn=bnn, q=q):
                    ifire(srcw_hbm, q + 2, si[bnn], ss[bnn])
                    ifire(dstw_hbm, q + 2, di[bnn], sd[bnn])
            return 0

        lax.fori_loop(0, C // 4, body_final, 0)
        plsc.subcore_barrier()
        pltpu.sync_copy(acc.at[pl.ds(row0, RS)],
                        part_hbm.at[cid, pl.ds(row0, RS)])

    return prop_kernel(v, srcw, dstw)


def _row_specs(R, D, n_wide, n_deg, out_wide):
    BR = R // 4
    wide = pl.BlockSpec((BR, D), lambda i: (i, 0))
    deg = pl.BlockSpec((BR, NL), lambda i: (i, 0))
    w = pl.BlockSpec(memory_space=pltpu.SMEM)
    return dict(
        grid=(4,),
        in_specs=[w] + [deg] * n_deg + [wide] * n_wide,
        out_specs=[wide] * out_wide if out_wide > 1 else wide,
    )


def _dinv(d0_ref, d1_ref):
    deg = d0_ref[:, 0:1] + d1_ref[:, 0:1] + 1.0
    return deg


def _prep_body(w_ref, d0, d1, x_ref, y_ref):
    y_ref[...] = lax.rsqrt(_dinv(d0, d1)) * x_ref[...]


def _first_body(w_ref, d0, d1, p0, p1, y, m_ref, v_ref):
    m = p0[...] + p1[...] + y[...]
    m_ref[...] = m
    v_ref[...] = (w_ref[10] / _dinv(d0, d1)) * m


def _step_body(w_ref, d0, d1, p0, p1, m, v, v_ref, *, k):
    z = w_ref[k] * m[...] + p0[...] + p1[...] + v[...]
    v_ref[...] = z / _dinv(d0, d1)


def _final_body(w_ref, d0, d1, p0, p1, m, v, x, o_ref):
    z = w_ref[1] * m[...] + p0[...] + p1[...] + v[...]
    o_ref[...] = w_ref[0] * x[...] + lax.rsqrt(_dinv(d0, d1)) * z


def _shape(R, D, n=1):
    s = jax.ShapeDtypeStruct((R, D), jnp.float32)
    return [s] * n if n > 1 else s


@jax.jit
def _prep(w, d0, d1, x):
    R, D = x.shape
    return pl.pallas_call(_prep_body, out_shape=_shape(R, D),
                          **_row_specs(R, D, 1, 2, 1))(w, d0, d1, x)


@jax.jit
def _first_combine(w, d0, d1, p0, p1, y):
    R, D = y.shape
    return pl.pallas_call(_first_body, out_shape=_shape(R, D, 2),
                          **_row_specs(R, D, 3, 2, 2))(w, d0, d1, p0, p1, y)


@functools.partial(jax.jit, static_argnames=("k",))
def _step_combine(w, d0, d1, p0, p1, m, v, *, k):
    R, D = v.shape
    return pl.pallas_call(functools.partial(_step_body, k=k),
                          out_shape=_shape(R, D),
                          **_row_specs(R, D, 4, 2, 1))(w, d0, d1, p0, p1, m, v)


@jax.jit
def _final_combine(w, d0, d1, p0, p1, m, v, x):
    R, D = x.shape
    return pl.pallas_call(_final_body, out_shape=_shape(R, D),
                          **_row_specs(R, D, 5, 2, 1))(w, d0, d1, p0, p1, m, v, x)


ORDER_K = 10


def kernel(x, edge_index, weights):
    N, D = x.shape
    E = edge_index.shape[1]
    R = -(-(N + 1) // (8 * NS)) * (8 * NS)
    C = -(-(-(-E // (NW * K))) // 4) * 4
    EP = NW * K * C

    src = edge_index[0].astype(jnp.int32)
    dst = edge_index[1].astype(jnp.int32)
    pad = jnp.full((EP - E,), N, jnp.int32)
    srcw = jnp.concatenate([src, pad]).reshape(NW, C, K)
    dstw = jnp.concatenate([dst, pad]).reshape(NW, C, K)
    xp = jnp.zeros((R, D), jnp.float32).at[:N].set(x)
    w = weights.astype(jnp.float32)

    degp = _deg_call(dstw, R=R, C=C)
    d0, d1 = degp[0], degp[1]
    y = _prep(w, d0, d1, xp)

    p = _prop_call(y, srcw, dstw, R=R, C=C)
    m, v = _first_combine(w, d0, d1, p[0], p[1], y)
    for k in range(ORDER_K - 1, 1, -1):
        p = _prop_call(v, srcw, dstw, R=R, C=C)
        v = _step_combine(w, d0, d1, p[0], p[1], m, v, k=k)
    p = _prop_call(v, srcw, dstw, R=R, C=C)
    out = _final_combine(w, d0, d1, p[0], p[1], m, v, xp)
    return out[:N]
